# baseline (speedup 1.0000x reference)
import functools

import jax
import jax.numpy as jnp
from jax import lax
from jax.experimental import pallas as pl
from jax.experimental.pallas import tpu as pltpu
from jax.experimental.pallas import tpu_sc as plsc

ENTITY_TOTAL = 100000
RELATION_TOTAL = 500
NREL = 2 * RELATION_TOTAL
DIM = 128
C = 5
B = 128

NG = 4
N_OCC = NG * B * 36
N_OUT = NG * B * 6
T = 64
KT = 16
NT = -(-(N_OCC // T + NREL) // KT) * KT
NSTEP = NT // KT
NW = 32

P1 = N_OUT // NW
P2C = 5
P2 = NG * B * 30 // NW // P2C
P3 = 2 * B // NW
P4 = (2 * B * 10 + 2 * B) // NW
PY = N_OUT // NW


def _wid():
    return lax.axis_index("s") * 2 + lax.axis_index("c")


_SC_MESH = plsc.VectorSubcoreMesh(core_axis_name="c", subcore_axis_name="s")


@functools.partial(
    pl.kernel,
    mesh=_SC_MESH,
    out_type=[
        jax.ShapeDtypeStruct((NT * T, DIM), jnp.float32),
        jax.ShapeDtypeStruct((2 * B * 12, DIM), jnp.float32),
    ],
    scratch_types=[
        pltpu.VMEM((1, P1), jnp.int32), pltpu.VMEM((1, P1), jnp.int32),
        pltpu.VMEM((P2C, P2), jnp.int32), pltpu.VMEM((P2C, P2), jnp.int32),
        pltpu.VMEM((1, P3), jnp.int32), pltpu.VMEM((1, P3), jnp.int32),
        pltpu.VMEM((1, P4), jnp.int32), pltpu.VMEM((1, P4), jnp.int32),
        pltpu.VMEM((P1, DIM), jnp.float32),
        pltpu.VMEM((P3, DIM), jnp.float32),
        pltpu.VMEM((P4, DIM), jnp.float32),
        pltpu.SemaphoreType.DMA,
    ],
)
def _sc_gather_kernel(ee, ect, re, rct,
                      i1, p1, i2, p2, i3, p3, i4, p4,
                      x_out, g_out,
                      bi1, bp1, bi2, bp2, bi3, bp3, bi4, bp4,
                      rows_a, rows_c, rows_d, sem):
    w = _wid()
    pltpu.sync_copy(i1.at[w], bi1)
    pltpu.sync_copy(p1.at[w], bp1)
    pltpu.sync_copy(i2.at[w], bi2)
    pltpu.sync_copy(p2.at[w], bp2)
    pltpu.sync_copy(i3.at[w], bi3)
    pltpu.sync_copy(p3.at[w], bp3)
    pltpu.sync_copy(i4.at[w], bi4)
    pltpu.sync_copy(p4.at[w], bp4)
    pltpu.async_copy(ee.at[bi1.at[0]], rows_a, sem).wait()
    pltpu.async_copy(rows_a, x_out.at[bp1.at[0]], sem).wait()
    for c in range(P2C):
        pltpu.async_copy(ect.at[bi2.at[c]], rows_a, sem).wait()
        pltpu.async_copy(rows_a, x_out.at[bp2.at[c]], sem).wait()
    pltpu.async_copy(re.at[bi3.at[0]], rows_c, sem).wait()
    pltpu.async_copy(rows_c, g_out.at[bp3.at[0]], sem).wait()
    pltpu.async_copy(rct.at[bi4.at[0]], rows_d, sem).wait()
    pltpu.async_copy(rows_d, g_out.at[bp4.at[0]], sem).wait()


@functools.partial(
    pl.kernel,
    mesh=_SC_MESH,
    out_type=jax.ShapeDtypeStruct((N_OUT, DIM), jnp.float32),
    scratch_types=[
        pltpu.VMEM((6, PY), jnp.int32),
        pltpu.VMEM((6, PY), jnp.float32),
        pltpu.VMEM((PY, DIM), jnp.float32), pltpu.VMEM((PY, DIM), jnp.float32),
        pltpu.VMEM((PY, DIM), jnp.float32), pltpu.VMEM((PY, DIM), jnp.float32),
        pltpu.VMEM((PY, DIM), jnp.float32), pltpu.VMEM((PY, DIM), jnp.float32),
        pltpu.VMEM((PY, DIM), jnp.float32),
        pltpu.SemaphoreType.DMA,
    ],
)
def _sc_reduce_kernel(y_pad, ypos, dres, out,
                      iy, ds_, b0, b1, b2, b3, b4, b5, ob, sem):
    w = _wid()
    pltpu.sync_copy(ypos.at[w], iy)
    pltpu.sync_copy(dres.at[w], ds_)
    pltpu.async_copy(y_pad.at[iy.at[0]], b0, sem).wait()
    pltpu.async_copy(y_pad.at[iy.at[1]], b1, sem).wait()
    pltpu.async_copy(y_pad.at[iy.at[2]], b2, sem).wait()
    pltpu.async_copy(y_pad.at[iy.at[3]], b3, sem).wait()
    pltpu.async_copy(y_pad.at[iy.at[4]], b4, sem).wait()
    pltpu.async_copy(y_pad.at[iy.at[5]], b5, sem).wait()

    def grp(i16, carry):
        base = i16 * 16
        dl = pl.ds(base, 16)
        dv0 = ds_[0, dl]
        dv1 = ds_[1, dl]
        dv2 = ds_[2, dl]
        dv3 = ds_[3, dl]
        dv4 = ds_[4, dl]
        dv5 = ds_[5, dl]
        for r in range(16):
            i = base + r
            for v in range(DIM // 16):
                s = pl.ds(v * 16, 16)
                acc = (dv0[r] * b0[i, s] + dv1[r] * b1[i, s]
                       + dv2[r] * b2[i, s] + dv3[r] * b3[i, s]
                       + dv4[r] * b4[i, s] + dv5[r] * b5[i, s])
                ob[i, s] = jnp.maximum(acc, 0.0)
        return carry

    lax.fori_loop(0, PY // 16, grp, 0)
    pltpu.sync_copy(ob, out.at[pl.ds(w * PY, PY)])


def _tc_matmul_body(tr_ref, x_ref, *wy_refs):
    y_ref = wy_refs[-1]
    for i in range(KT):
        r = pl.ds(i * T, T)
        y_ref[r, :] = jnp.dot(x_ref[r, :].astype(jnp.bfloat16),
                              wy_refs[i][0].astype(jnp.bfloat16),
                              preferred_element_type=jnp.float32)


def _tc_rel_matmul(tile_rel, x_pad, weight):
    w_specs = [
        pl.BlockSpec((1, DIM, DIM),
                     lambda t, s, i=i: (s[t * KT + i], 0, 0))
        for i in range(KT)
    ]
    grid_spec = pltpu.PrefetchScalarGridSpec(
        num_scalar_prefetch=1,
        grid=(NSTEP,),
        in_specs=[pl.BlockSpec((KT * T, DIM), lambda t, s: (t, 0))] + w_specs,
        out_specs=pl.BlockSpec((KT * T, DIM), lambda t, s: (t, 0)),
    )
    return pl.pallas_call(
        _tc_matmul_body,
        grid_spec=grid_spec,
        out_shape=jax.ShapeDtypeStruct((NT * T, DIM), jnp.float32),
        compiler_params=pltpu.CompilerParams(
            dimension_semantics=("arbitrary",)),
    )(tile_rel, x_pad, *([weight] * KT))


def _tc_gcn_body(a_ref, g_ref, w_ref, o_ref):
    h = g_ref[:, 0:6, :] + g_ref[:, 6:12, :]
    a = a_ref[:, :6, :6]
    s = lax.dot_general(a, h, (((2,), (1,)), ((0,), (0,))),
                        preferred_element_type=jnp.float32)
    s2 = s.reshape(16 * 6, DIM)
    o = jnp.dot(s2, w_ref[...], preferred_element_type=jnp.float32)
    o_ref[...] = jnp.maximum(o, 0.0).reshape(16, 6, DIM)


def _tc_gcn(a_pad, g, w_rel):
    return pl.pallas_call(
        _tc_gcn_body,
        grid=(2 * B // 16,),
        in_specs=[
            pl.BlockSpec((16, 8, 8), lambda i: (i, 0, 0)),
            pl.BlockSpec((16, 12, DIM), lambda i: (i, 0, 0)),
            pl.BlockSpec((DIM, DIM), lambda i: (0, 0)),
        ],
        out_specs=pl.BlockSpec((16, 6, DIM), lambda i: (i, 0, 0)),
        out_shape=jax.ShapeDtypeStruct((2 * B, 6, DIM), jnp.float32),
    )(a_pad, g.reshape(2 * B, 12, DIM), w_rel)


def kernel(epoch, pos_h, pos_r, pos_t, neg_h, neg_r, neg_t, ph_R, ph_D, ph_nn, pr_A, pt_R, pt_D, pt_nn, nh_R, nh_D, nh_nn, nr_A, nt_R, nt_D, nt_nn, entity_emb, relation_emb, entity_context_table, relation_context_table, entity_gcn_weight, relation_gcn_weight, entity_adj_table, relation_adj_table):
    i32 = jnp.int32
    ids_all = jnp.stack([pos_h, pos_t, neg_h, neg_t]).astype(i32)
    eadj = entity_adj_table[ids_all].astype(i32)
    R_all = jnp.stack([ph_R, pt_R, nh_R, nt_R]).astype(i32)
    rel_flat = R_all.reshape(N_OCC)
    skey = jnp.sort(rel_flat * 32768 + jnp.arange(N_OCC, dtype=i32))
    order = skey & 32767
    rel_sorted = skey >> 15
    counts = jnp.bincount(rel_flat, length=NREL)
    ntiles = (counts + T - 1) // T
    tile_cum = jnp.cumsum(ntiles)
    row_start = jnp.concatenate([jnp.zeros(1, i32),
                                 tile_cum.astype(i32)]) * T
    seg_start = jnp.concatenate([jnp.zeros(1, i32),
                                 jnp.cumsum(counts).astype(i32)])
    delta = row_start - seg_start
    padpos = delta[rel_sorted] + jnp.arange(N_OCC, dtype=i32)
    scatter_pos = jnp.zeros(N_OCC, i32).at[order].set(padpos)
    marks = jnp.zeros(NT, i32).at[tile_cum].add(1, mode='drop')
    tile_rel = jnp.cumsum(marks).astype(i32)
    D_all = jnp.stack([ph_D, pt_D, nh_D, nt_D]).reshape(N_OCC)
    d_res = (D_all.reshape(N_OUT, 6).T
             .reshape(6, NW, PY).transpose(1, 0, 2))

    sp4 = scatter_pos.reshape(NG, B, 6, 6)
    pos1 = sp4[..., 0].reshape(NW, 1, P1)
    idx1 = jnp.broadcast_to(ids_all[:, :, None], (NG, B, 6)).reshape(NW, 1, P1)
    pos2 = sp4[..., 1:].reshape(NW, P2C, P2)
    idx2 = jnp.broadcast_to(eadj[:, :, None, :],
                            (NG, B, 6, C)).reshape(NW, P2C, P2)

    rids = jnp.stack([pos_r, neg_r]).reshape(2 * B).astype(i32)
    radj = relation_adj_table[rids].astype(i32)
    s2 = jnp.arange(2 * B, dtype=i32)
    pos3 = (s2 * 12).reshape(NW, 1, P3)
    idx3 = rids.reshape(NW, 1, P3)
    m = jnp.arange(2 * C, dtype=i32)
    gslot = jnp.where(m % 2 == 0, 1 + m // 2, 7 + m // 2)
    gpos = (s2[:, None] * 12 + gslot[None, :]).reshape(2 * B * 2 * C)
    posz = s2 * 12 + 6
    idxz = jnp.full(2 * B, RELATION_TOTAL, i32)
    pos4 = jnp.concatenate([gpos, posz]).reshape(NW, 1, P4)
    idx4 = jnp.concatenate([radj.reshape(2 * B * 2 * C), idxz]).reshape(NW, 1, P4)

    ypos = (scatter_pos.reshape(N_OUT, 6).T
            .reshape(6, NW, PY).transpose(1, 0, 2))

    a_all = jnp.stack([pr_A, nr_A]).reshape(2 * B, 6, 6)
    a_pad = jnp.zeros((2 * B, 8, 8), jnp.float32).at[:, :6, :6].set(a_all)

    x_pad, g = _sc_gather_kernel(
        entity_emb, entity_context_table, relation_emb,
        relation_context_table, idx1, pos1, idx2, pos2, idx3, pos3,
        idx4, pos4)
    y_pad = _tc_rel_matmul(tile_rel, x_pad, entity_gcn_weight)
    rg = _sc_reduce_kernel(y_pad, ypos, d_res)
    rg = rg.reshape(NG, B, 6, DIM)
    gcn_out = _tc_gcn(a_pad, g, relation_gcn_weight)
    return (rg[0], rg[1], rg[2], rg[3], gcn_out[:B], gcn_out[B:])


# speedup vs baseline: 1.0113x; 1.0113x over previous
import functools

import jax
import jax.numpy as jnp
from jax import lax
from jax.experimental import pallas as pl
from jax.experimental.pallas import tpu as pltpu
from jax.experimental.pallas import tpu_sc as plsc

ENTITY_TOTAL = 100000
RELATION_TOTAL = 500
NREL = 2 * RELATION_TOTAL
DIM = 128
C = 5
B = 128

NG = 4
N_OCC = NG * B * 36
N_OUT = NG * B * 6
T = 64
KT = 16
NT = -(-(N_OCC // T + NREL) // KT) * KT
NSTEP = NT // KT
NW = 32

P1 = N_OUT // NW
P2C = 5
P2 = NG * B * 30 // NW // P2C
P3 = 2 * B // NW
P4 = (2 * B * 10 + 2 * B) // NW
PY = N_OUT // NW


def _wid():
    return lax.axis_index("s") * 2 + lax.axis_index("c")


_SC_MESH = plsc.VectorSubcoreMesh(core_axis_name="c", subcore_axis_name="s")


@functools.partial(
    pl.kernel,
    mesh=_SC_MESH,
    out_type=[
        jax.ShapeDtypeStruct((NT * T, DIM), jnp.float32),
        jax.ShapeDtypeStruct((2 * B * 12, DIM), jnp.float32),
    ],
    scratch_types=[
        pltpu.VMEM((1, P1), jnp.int32), pltpu.VMEM((1, P1), jnp.int32),
        pltpu.VMEM((P2C, P2), jnp.int32), pltpu.VMEM((P2C, P2), jnp.int32),
        pltpu.VMEM((1, P3), jnp.int32), pltpu.VMEM((1, P3), jnp.int32),
        pltpu.VMEM((1, P4), jnp.int32), pltpu.VMEM((1, P4), jnp.int32),
        pltpu.VMEM((P1, DIM), jnp.float32),
        pltpu.VMEM((P2, DIM), jnp.float32), pltpu.VMEM((P2, DIM), jnp.float32),
        pltpu.VMEM((P2, DIM), jnp.float32), pltpu.VMEM((P2, DIM), jnp.float32),
        pltpu.VMEM((P2, DIM), jnp.float32),
        pltpu.VMEM((P3, DIM), jnp.float32),
        pltpu.VMEM((P4, DIM), jnp.float32),
        pltpu.SemaphoreType.DMA,
    ],
)
def _sc_gather_kernel(ee, ect, re, rct,
                      i1, p1, i2, p2, i3, p3, i4, p4,
                      x_out, g_out,
                      bi1, bp1, bi2, bp2, bi3, bp3, bi4, bp4,
                      ra, rb0, rb1, rb2, rb3, rb4, rows_c, rows_d, sem):
    w = _wid()
    pltpu.sync_copy(i1.at[w], bi1)
    pltpu.sync_copy(p1.at[w], bp1)
    pltpu.sync_copy(i2.at[w], bi2)
    pltpu.sync_copy(p2.at[w], bp2)
    pltpu.sync_copy(i3.at[w], bi3)
    pltpu.sync_copy(p3.at[w], bp3)
    pltpu.sync_copy(i4.at[w], bi4)
    pltpu.sync_copy(p4.at[w], bp4)
    rbufs = [rb0, rb1, rb2, rb3, rb4]
    gathers = [pltpu.async_copy(ee.at[bi1.at[0]], ra, sem)]
    for c in range(P2C):
        gathers.append(pltpu.async_copy(ect.at[bi2.at[c]], rbufs[c], sem))
    gathers.append(pltpu.async_copy(re.at[bi3.at[0]], rows_c, sem))
    gathers.append(pltpu.async_copy(rct.at[bi4.at[0]], rows_d, sem))
    for gth in gathers:
        gth.wait()
    scatters = [pltpu.async_copy(ra, x_out.at[bp1.at[0]], sem)]
    for c in range(P2C):
        scatters.append(
            pltpu.async_copy(rbufs[c], x_out.at[bp2.at[c]], sem))
    scatters.append(pltpu.async_copy(rows_c, g_out.at[bp3.at[0]], sem))
    scatters.append(pltpu.async_copy(rows_d, g_out.at[bp4.at[0]], sem))
    for sct in scatters:
        sct.wait()


@functools.partial(
    pl.kernel,
    mesh=_SC_MESH,
    out_type=jax.ShapeDtypeStruct((N_OUT, DIM), jnp.float32),
    scratch_types=[
        pltpu.VMEM((6, PY), jnp.int32),
        pltpu.VMEM((6, PY), jnp.float32),
        pltpu.VMEM((PY, DIM), jnp.float32), pltpu.VMEM((PY, DIM), jnp.float32),
        pltpu.VMEM((PY, DIM), jnp.float32), pltpu.VMEM((PY, DIM), jnp.float32),
        pltpu.VMEM((PY, DIM), jnp.float32), pltpu.VMEM((PY, DIM), jnp.float32),
        pltpu.VMEM((PY, DIM), jnp.float32),
        pltpu.SemaphoreType.DMA,
    ],
)
def _sc_reduce_kernel(y_pad, ypos, dres, out,
                      iy, ds_, b0, b1, b2, b3, b4, b5, ob, sem):
    w = _wid()
    pltpu.sync_copy(ypos.at[w], iy)
    pltpu.sync_copy(dres.at[w], ds_)
    pltpu.async_copy(y_pad.at[iy.at[0]], b0, sem).wait()
    pltpu.async_copy(y_pad.at[iy.at[1]], b1, sem).wait()
    pltpu.async_copy(y_pad.at[iy.at[2]], b2, sem).wait()
    pltpu.async_copy(y_pad.at[iy.at[3]], b3, sem).wait()
    pltpu.async_copy(y_pad.at[iy.at[4]], b4, sem).wait()
    pltpu.async_copy(y_pad.at[iy.at[5]], b5, sem).wait()

    def grp(i16, carry):
        base = i16 * 16
        dl = pl.ds(base, 16)
        dv0 = ds_[0, dl]
        dv1 = ds_[1, dl]
        dv2 = ds_[2, dl]
        dv3 = ds_[3, dl]
        dv4 = ds_[4, dl]
        dv5 = ds_[5, dl]
        for r in range(16):
            i = base + r
            for v in range(DIM // 16):
                s = pl.ds(v * 16, 16)
                acc = (dv0[r] * b0[i, s] + dv1[r] * b1[i, s]
                       + dv2[r] * b2[i, s] + dv3[r] * b3[i, s]
                       + dv4[r] * b4[i, s] + dv5[r] * b5[i, s])
                ob[i, s] = jnp.maximum(acc, 0.0)
        return carry

    lax.fori_loop(0, PY // 16, grp, 0)
    pltpu.sync_copy(ob, out.at[pl.ds(w * PY, PY)])


def _tc_matmul_body(tr_ref, x_ref, *wy_refs):
    y_ref = wy_refs[-1]
    for i in range(KT):
        r = pl.ds(i * T, T)
        y_ref[r, :] = jnp.dot(x_ref[r, :], wy_refs[i][0],
                              preferred_element_type=jnp.float32)


def _tc_rel_matmul(tile_rel, x_pad, weight):
    w_specs = [
        pl.BlockSpec((1, DIM, DIM),
                     lambda t, s, i=i: (s[t * KT + i], 0, 0))
        for i in range(KT)
    ]
    grid_spec = pltpu.PrefetchScalarGridSpec(
        num_scalar_prefetch=1,
        grid=(NSTEP,),
        in_specs=[pl.BlockSpec((KT * T, DIM), lambda t, s: (t, 0))] + w_specs,
        out_specs=pl.BlockSpec((KT * T, DIM), lambda t, s: (t, 0)),
    )
    return pl.pallas_call(
        _tc_matmul_body,
        grid_spec=grid_spec,
        out_shape=jax.ShapeDtypeStruct((NT * T, DIM), jnp.float32),
        compiler_params=pltpu.CompilerParams(
            dimension_semantics=("arbitrary",)),
    )(tile_rel, x_pad, *([weight] * KT))


def _tc_gcn_body(a_ref, g_ref, w_ref, o_ref):
    h = g_ref[:, 0:6, :] + g_ref[:, 6:12, :]
    a = a_ref[:, :6, :6]
    s = lax.dot_general(a, h, (((2,), (1,)), ((0,), (0,))),
                        preferred_element_type=jnp.float32)
    s2 = s.reshape(16 * 6, DIM)
    o = jnp.dot(s2, w_ref[...], preferred_element_type=jnp.float32)
    o_ref[...] = jnp.maximum(o, 0.0).reshape(16, 6, DIM)


def _tc_gcn(a_pad, g, w_rel):
    return pl.pallas_call(
        _tc_gcn_body,
        grid=(2 * B // 16,),
        in_specs=[
            pl.BlockSpec((16, 8, 8), lambda i: (i, 0, 0)),
            pl.BlockSpec((16, 12, DIM), lambda i: (i, 0, 0)),
            pl.BlockSpec((DIM, DIM), lambda i: (0, 0)),
        ],
        out_specs=pl.BlockSpec((16, 6, DIM), lambda i: (i, 0, 0)),
        out_shape=jax.ShapeDtypeStruct((2 * B, 6, DIM), jnp.float32),
    )(a_pad, g.reshape(2 * B, 12, DIM), w_rel)


def kernel(epoch, pos_h, pos_r, pos_t, neg_h, neg_r, neg_t, ph_R, ph_D, ph_nn, pr_A, pt_R, pt_D, pt_nn, nh_R, nh_D, nh_nn, nr_A, nt_R, nt_D, nt_nn, entity_emb, relation_emb, entity_context_table, relation_context_table, entity_gcn_weight, relation_gcn_weight, entity_adj_table, relation_adj_table):
    i32 = jnp.int32
    ids_all = jnp.stack([pos_h, pos_t, neg_h, neg_t]).astype(i32)
    eadj = entity_adj_table[ids_all].astype(i32)
    R_all = jnp.stack([ph_R, pt_R, nh_R, nt_R]).astype(i32)
    rel_flat = R_all.reshape(N_OCC)
    skey = jnp.sort(rel_flat * 32768 + jnp.arange(N_OCC, dtype=i32))
    order = skey & 32767
    rel_sorted = skey >> 15
    counts = jnp.bincount(rel_flat, length=NREL)
    ntiles = (counts + T - 1) // T
    tile_cum = jnp.cumsum(ntiles)
    row_start = jnp.concatenate([jnp.zeros(1, i32),
                                 tile_cum.astype(i32)]) * T
    seg_start = jnp.concatenate([jnp.zeros(1, i32),
                                 jnp.cumsum(counts).astype(i32)])
    delta = row_start - seg_start
    padpos = delta[rel_sorted] + jnp.arange(N_OCC, dtype=i32)
    scatter_pos = jnp.zeros(N_OCC, i32).at[order].set(padpos)
    marks = jnp.zeros(NT, i32).at[tile_cum].add(1, mode='drop')
    tile_rel = jnp.cumsum(marks).astype(i32)
    D_all = jnp.stack([ph_D, pt_D, nh_D, nt_D]).reshape(N_OCC)
    d_res = (D_all.reshape(N_OUT, 6).T
             .reshape(6, NW, PY).transpose(1, 0, 2))

    sp4 = scatter_pos.reshape(NG, B, 6, 6)
    pos1 = sp4[..., 0].reshape(NW, 1, P1)
    idx1 = jnp.broadcast_to(ids_all[:, :, None], (NG, B, 6)).reshape(NW, 1, P1)
    pos2 = sp4[..., 1:].reshape(NW, P2C, P2)
    idx2 = jnp.broadcast_to(eadj[:, :, None, :],
                            (NG, B, 6, C)).reshape(NW, P2C, P2)

    rids = jnp.stack([pos_r, neg_r]).reshape(2 * B).astype(i32)
    radj = relation_adj_table[rids].astype(i32)
    s2 = jnp.arange(2 * B, dtype=i32)
    pos3 = (s2 * 12).reshape(NW, 1, P3)
    idx3 = rids.reshape(NW, 1, P3)
    m = jnp.arange(2 * C, dtype=i32)
    gslot = jnp.where(m % 2 == 0, 1 + m // 2, 7 + m // 2)
    gpos = (s2[:, None] * 12 + gslot[None, :]).reshape(2 * B * 2 * C)
    posz = s2 * 12 + 6
    idxz = jnp.full(2 * B, RELATION_TOTAL, i32)
    pos4 = jnp.concatenate([gpos, posz]).reshape(NW, 1, P4)
    idx4 = jnp.concatenate([radj.reshape(2 * B * 2 * C), idxz]).reshape(NW, 1, P4)

    ypos = (scatter_pos.reshape(N_OUT, 6).T
            .reshape(6, NW, PY).transpose(1, 0, 2))

    a_all = jnp.stack([pr_A, nr_A]).reshape(2 * B, 6, 6)
    a_pad = jnp.zeros((2 * B, 8, 8), jnp.float32).at[:, :6, :6].set(a_all)

    x_pad, g = _sc_gather_kernel(
        entity_emb, entity_context_table, relation_emb,
        relation_context_table, idx1, pos1, idx2, pos2, idx3, pos3,
        idx4, pos4)
    y_pad = _tc_rel_matmul(tile_rel, x_pad, entity_gcn_weight)
    rg = _sc_reduce_kernel(y_pad, ypos, d_res)
    rg = rg.reshape(NG, B, 6, DIM)
    gcn_out = _tc_gcn(a_pad, g, relation_gcn_weight)
    return (rg[0], rg[1], rg[2], rg[3], gcn_out[:B], gcn_out[B:])
